# baseline (speedup 1.0000x reference)
import functools

import jax
import jax.numpy as jnp
from jax import lax
from jax.experimental import pallas as pl
from jax.experimental.pallas import tpu as pltpu
from jax.experimental.pallas import tpu_sc as plsc

_NC = 2
_NS = 16

_GD = lax.GatherDimensionNumbers(
    offset_dims=(), collapsed_slice_dims=(0,), start_index_map=(0,)
)


def _lane(v, t):
    idx = jnp.full((16, 1), t, jnp.int32)
    return lax.gather(
        v, idx, dimension_numbers=_GD, slice_sizes=(1,),
        mode=lax.GatherScatterMode.PROMISE_IN_BOUNDS,
    )


def _seg_sum_sc(rows_flat, idx, zeros, n_out, E):
    NH = n_out // 2
    e_per_c = E // _NC
    CH = 128
    n_chunks = e_per_c // CH
    mesh = plsc.VectorSubcoreMesh(core_axis_name="c", subcore_axis_name="s")

    @functools.partial(
        pl.kernel,
        mesh=mesh,
        compiler_params=pltpu.CompilerParams(needs_layout_passes=False),
        out_type=jax.ShapeDtypeStruct((_NC * 8 * n_out * 16,), jnp.float32),
        scratch_types=[
            pltpu.VMEM((CH,), jnp.int32),
            pltpu.VMEM((CH * 16,), jnp.float32),
            pltpu.VMEM((NH * 16,), jnp.float32),
        ],
    )
    def k(rows_hbm, idx_hbm, zeros_hbm, out_hbm, idx_v, rows_v, tab_v):
        c = lax.axis_index("c")
        s = lax.axis_index("s")
        nh = s // 8
        g = s % 8
        lo = nh * NH
        pltpu.sync_copy(zeros_hbm.at[pl.ds(0, NH * 16)], tab_v)
        base = c * e_per_c
        iota16 = lax.iota(jnp.int32, 16)
        nh_c = jnp.int32(NH)

        def chunk_body(j, carry):
            off = base + j * CH
            pltpu.sync_copy(idx_hbm.at[pl.ds(off, CH)], idx_v)
            pltpu.sync_copy(
                rows_hbm.at[pl.ds((g * E + off) * 16, CH * 16)], rows_v
            )
            for j2 in range(CH // 16):
                rel = idx_v[pl.ds(j2 * 16, 16)] - lo
                for t in range(16):
                    brel = _lane(rel, t)
                    mask = (brel >= 0) & (brel < nh_c)
                    vals = rows_v[pl.ds((j2 * 16 + t) * 16, 16)]
                    addrs = brel * 16 + iota16
                    plsc.addupdate_scatter(tab_v, [addrs], vals, mask=mask)
            return carry

        lax.fori_loop(0, n_chunks, chunk_body, 0)
        pltpu.sync_copy(
            tab_v,
            out_hbm.at[pl.ds(((c * 8 + g) * n_out + lo) * 16, NH * 16)],
        )

    return k(rows_flat, idx, zeros)


def _seg_sum_rows(rows, idx, zeros, n_out):
    E = rows.shape[0]
    rows_t = rows.reshape(E, 8, 16).transpose(1, 0, 2).reshape(-1)
    out = _seg_sum_sc(rows_t, idx, zeros, n_out, E)
    out = out.reshape(_NC, 8, n_out, 16)
    comb = out[0] + out[1]
    return comb.transpose(1, 0, 2).reshape(n_out, 128)


def _bn(x, g, b):
    m = x.mean(0)
    v = x.var(0)
    return (x - m) / jnp.sqrt(v + 1e-5) * g + b


def kernel(x_all, edge_dts, params, node_ids, edge_index, batch_size):
    src = edge_index[0]
    dst = edge_index[1]
    n_sub = node_ids.shape[0]
    E = edge_dts.shape[0]
    x = x_all[node_ids]
    ones = jnp.ones((E,), jnp.float32)
    out_deg = jax.ops.segment_sum(ones, src, num_segments=n_sub)
    in_deg = jax.ops.segment_sum(ones, dst, num_segments=n_sub)
    deg_ratio = out_deg / (in_deg + 1.0)
    min_dts = jnp.full((n_sub,), 1e9, jnp.float32).at[dst].min(edge_dts)
    max_dts = jnp.zeros((n_sub,), jnp.float32).at[dst].max(edge_dts)
    recency = jnp.minimum(min_dts, 1e8)
    activity_window = jnp.maximum(max_dts - min_dts, 1.0)
    burst_cutoff = min_dts + 0.25 * activity_window
    is_burst = (edge_dts <= burst_cutoff[dst]).astype(jnp.float32)
    burst_count = jax.ops.segment_sum(is_burst, dst, num_segments=n_sub)
    burst_ratio = burst_count / jnp.maximum(in_deg, 1.0)
    s1 = jax.ops.segment_sum(edge_dts, dst, num_segments=n_sub)
    s2 = jax.ops.segment_sum(edge_dts ** 2, dst, num_segments=n_sub)
    cnt = jnp.maximum(in_deg, 1.0)
    mean_dts = jnp.where(in_deg > 0, s1 / cnt, 0.0)
    dts_sq = jnp.where(in_deg > 0, s2 / cnt, 0.0)
    std_dts = jnp.sqrt(jnp.maximum(dts_sq - mean_dts ** 2, 0.0))
    extra = jnp.stack([out_deg, in_deg, deg_ratio, recency, burst_ratio, mean_dts, std_dts], axis=1)
    B = 2000
    bsf = jnp.asarray(batch_size).astype(jnp.float32)
    mu = extra[:B].sum(0) / bsf
    sd = jnp.maximum(jnp.std(extra[:B], axis=0, ddof=1), 1e-8)
    extra = (extra - mu) / sd
    x = jnp.concatenate([x, extra], axis=1)
    rel_t = jnp.cos(edge_dts[:, None] * params['basis_freq'][None, :] + params['phase'][None, :])
    h = jax.nn.relu(x @ params['proj_W'] + params['proj_b'])
    H = 8
    C = h.shape[1] // H
    zeros_tab = jnp.zeros((n_sub * 16 // 2,), jnp.float32)
    for lp in params['layers']:
        q = (h @ lp['Wq'] + lp['bq']).reshape(n_sub, H, C)
        k = (h @ lp['Wk'] + lp['bk']).reshape(n_sub, H, C)
        v = (h @ lp['Wv'] + lp['bv']).reshape(n_sub, H, C)
        e = (rel_t @ lp['We']).reshape(E, H, C)
        kj = k[src] + e
        vj = v[src] + e
        alpha = (q[dst] * kj).sum(-1) / jnp.sqrt(float(C))
        amax = jax.ops.segment_max(alpha, dst, num_segments=n_sub)
        amax = jnp.where(jnp.isfinite(amax), amax, 0.0)
        ex = jnp.exp(alpha - amax[dst])
        den = jax.ops.segment_sum(ex, dst, num_segments=n_sub)
        w = ex / jnp.maximum(den[dst], 1e-16)
        wvj = (w[:, :, None] * vj).reshape(E, H * C)
        out = _seg_sum_rows(wvj, dst, zeros_tab, n_sub)
        out = out + h @ lp['Ws'] + lp['bs']
        h = _bn(jax.nn.relu(out), lp['bn_g'], lp['bn_b'])
    z = h[:B]
    c = params['clf']
    z = jax.nn.relu(_bn(z @ c['W1'] + c['b1'], c['g1'], c['be1']))
    z = jax.nn.relu(_bn(z @ c['W2'] + c['b2'], c['g2'], c['be2']))
    return (z @ c['W3'] + c['b3']).squeeze(-1)


# speedup vs baseline: 1.0709x; 1.0709x over previous
import functools

import jax
import jax.numpy as jnp
from jax import lax
from jax.experimental import pallas as pl
from jax.experimental.pallas import tpu as pltpu
from jax.experimental.pallas import tpu_sc as plsc

_NC = 2
_NS = 16

_GD = lax.GatherDimensionNumbers(
    offset_dims=(), collapsed_slice_dims=(0,), start_index_map=(0,)
)


def _lane(v, t):
    idx = jnp.full((16, 1), t, jnp.int32)
    return lax.gather(
        v, idx, dimension_numbers=_GD, slice_sizes=(1,),
        mode=lax.GatherScatterMode.PROMISE_IN_BOUNDS,
    )


def _seg_sum_sc(rows_flat, idx, zeros, n_out, E):
    NH = n_out // 2
    e_per_c = E // _NC
    CH = 320
    n_chunks = e_per_c // CH
    mesh = plsc.VectorSubcoreMesh(core_axis_name="c", subcore_axis_name="s")

    @functools.partial(
        pl.kernel,
        mesh=mesh,
        compiler_params=pltpu.CompilerParams(needs_layout_passes=False),
        out_type=jax.ShapeDtypeStruct((_NC * 8 * n_out * 16,), jnp.float32),
        scratch_types=[
            pltpu.VMEM((CH,), jnp.int32),
            pltpu.VMEM((CH * 16,), jnp.float32),
            pltpu.VMEM((NH * 16,), jnp.float32),
            pltpu.SemaphoreType.DMA,
            pltpu.SemaphoreType.DMA,
        ],
    )
    def k(rows_hbm, idx_hbm, zeros_hbm, out_hbm, idx_v, rows_v, tab_v,
          sem_i, sem_r):
        c = lax.axis_index("c")
        s = lax.axis_index("s")
        nh = s // 8
        g = s % 8
        lo = nh * NH
        pltpu.sync_copy(zeros_hbm.at[pl.ds(0, NH * 16)], tab_v)
        base = c * e_per_c
        iota16 = lax.iota(jnp.int32, 16)
        nh_c = jnp.int32(NH)

        def chunk_body(j, carry):
            off = base + j * CH
            di = pltpu.async_copy(idx_hbm.at[pl.ds(off, CH)], idx_v, sem_i)
            dr = pltpu.async_copy(
                rows_hbm.at[pl.ds((g * E + off) * 16, CH * 16)], rows_v,
                sem_r,
            )
            di.wait()
            dr.wait()
            for j2 in range(CH // 16):
                rel = idx_v[pl.ds(j2 * 16, 16)] - lo
                for t in range(16):
                    brel = _lane(rel, t)
                    mask = (brel >= 0) & (brel < nh_c)
                    vals = rows_v[pl.ds((j2 * 16 + t) * 16, 16)]
                    addrs = brel * 16 + iota16
                    plsc.addupdate_scatter(tab_v, [addrs], vals, mask=mask)
            return carry

        lax.fori_loop(0, n_chunks, chunk_body, 0)
        pltpu.sync_copy(
            tab_v,
            out_hbm.at[pl.ds(((c * 8 + g) * n_out + lo) * 16, NH * 16)],
        )

    return k(rows_flat, idx, zeros)


def _seg_sum_rows(rows, idx, zeros, n_out):
    E = rows.shape[0]
    rows_t = rows.reshape(E, 8, 16).transpose(1, 0, 2).reshape(-1)
    out = _seg_sum_sc(rows_t, idx, zeros, n_out, E)
    out = out.reshape(_NC, 8, n_out, 16)
    comb = out[0] + out[1]
    return comb.transpose(1, 0, 2).reshape(n_out, 128)


def _bn(x, g, b):
    m = x.mean(0)
    v = x.var(0)
    return (x - m) / jnp.sqrt(v + 1e-5) * g + b


def kernel(x_all, edge_dts, params, node_ids, edge_index, batch_size):
    src = edge_index[0]
    dst = edge_index[1]
    n_sub = node_ids.shape[0]
    E = edge_dts.shape[0]
    x = x_all[node_ids]
    ones = jnp.ones((E,), jnp.float32)
    out_deg = jax.ops.segment_sum(ones, src, num_segments=n_sub)
    in_deg = jax.ops.segment_sum(ones, dst, num_segments=n_sub)
    deg_ratio = out_deg / (in_deg + 1.0)
    min_dts = jnp.full((n_sub,), 1e9, jnp.float32).at[dst].min(edge_dts)
    max_dts = jnp.zeros((n_sub,), jnp.float32).at[dst].max(edge_dts)
    recency = jnp.minimum(min_dts, 1e8)
    activity_window = jnp.maximum(max_dts - min_dts, 1.0)
    burst_cutoff = min_dts + 0.25 * activity_window
    is_burst = (edge_dts <= burst_cutoff[dst]).astype(jnp.float32)
    burst_count = jax.ops.segment_sum(is_burst, dst, num_segments=n_sub)
    burst_ratio = burst_count / jnp.maximum(in_deg, 1.0)
    s1 = jax.ops.segment_sum(edge_dts, dst, num_segments=n_sub)
    s2 = jax.ops.segment_sum(edge_dts ** 2, dst, num_segments=n_sub)
    cnt = jnp.maximum(in_deg, 1.0)
    mean_dts = jnp.where(in_deg > 0, s1 / cnt, 0.0)
    dts_sq = jnp.where(in_deg > 0, s2 / cnt, 0.0)
    std_dts = jnp.sqrt(jnp.maximum(dts_sq - mean_dts ** 2, 0.0))
    extra = jnp.stack([out_deg, in_deg, deg_ratio, recency, burst_ratio, mean_dts, std_dts], axis=1)
    B = 2000
    bsf = jnp.asarray(batch_size).astype(jnp.float32)
    mu = extra[:B].sum(0) / bsf
    sd = jnp.maximum(jnp.std(extra[:B], axis=0, ddof=1), 1e-8)
    extra = (extra - mu) / sd
    x = jnp.concatenate([x, extra], axis=1)
    rel_t = jnp.cos(edge_dts[:, None] * params['basis_freq'][None, :] + params['phase'][None, :])
    h = jax.nn.relu(x @ params['proj_W'] + params['proj_b'])
    H = 8
    C = h.shape[1] // H
    zeros_tab = jnp.zeros((n_sub * 16 // 2,), jnp.float32)
    for lp in params['layers']:
        q = (h @ lp['Wq'] + lp['bq']).reshape(n_sub, H, C)
        k = (h @ lp['Wk'] + lp['bk']).reshape(n_sub, H, C)
        v = (h @ lp['Wv'] + lp['bv']).reshape(n_sub, H, C)
        e = (rel_t @ lp['We']).reshape(E, H, C)
        kj = k[src] + e
        vj = v[src] + e
        alpha = (q[dst] * kj).sum(-1) / jnp.sqrt(float(C))
        amax = jax.ops.segment_max(alpha, dst, num_segments=n_sub)
        amax = jnp.where(jnp.isfinite(amax), amax, 0.0)
        ex = jnp.exp(alpha - amax[dst])
        den = jax.ops.segment_sum(ex, dst, num_segments=n_sub)
        w = ex / jnp.maximum(den[dst], 1e-16)
        wvj = (w[:, :, None] * vj).reshape(E, H * C)
        out = _seg_sum_rows(wvj, dst, zeros_tab, n_sub)
        out = out + h @ lp['Ws'] + lp['bs']
        h = _bn(jax.nn.relu(out), lp['bn_g'], lp['bn_b'])
    z = h[:B]
    c = params['clf']
    z = jax.nn.relu(_bn(z @ c['W1'] + c['b1'], c['g1'], c['be1']))
    z = jax.nn.relu(_bn(z @ c['W2'] + c['b2'], c['g2'], c['be2']))
    return (z @ c['W3'] + c['b3']).squeeze(-1)
